# baseline (speedup 1.0000x reference)
import functools
import math

import jax
import jax.numpy as jnp
from jax import lax
from jax.experimental import pallas as pl
from jax.experimental.pallas import tpu as pltpu
from jax.experimental.pallas import tpu_sc as plsc

BATCH = 4
SEQ_LEN = 8192
EMBED_DIM = 64
VOCAB = 1000000
HALF_V = VOCAB // 2
MAX_WAVELENGTH = 10000.0

_NC = 2
_NS = 16
_NW = _NC * _NS
_ROWS = BATCH * SEQ_LEN
_BW = _ROWS // _NW
_CH = 256
_NCHUNK = _BW // _CH
_LANES = EMBED_DIM // 16

_TCB = 1024


def _relayout_body(in_ref, out_ref):
    x = in_ref[...]
    r = lax.broadcasted_iota(jnp.int32, (EMBED_DIM, EMBED_DIM), 0)
    c = lax.broadcasted_iota(jnp.int32, (EMBED_DIM, EMBED_DIM), 1)
    ident = (r == c).astype(jnp.float32)
    xt = lax.dot_general(
        x, ident, (((0,), (0,)), ((), ())), preferred_element_type=jnp.float32
    )
    out_ref[...] = jnp.concatenate([xt[:_TCB], xt[_TCB:]], axis=1)


_RGRID = (VOCAB + 2 * _TCB - 1) // (2 * _TCB)


def _relayout_tc(tabt):
    return pl.pallas_call(
        _relayout_body,
        grid=(_RGRID,),
        in_specs=[pl.BlockSpec((EMBED_DIM, 2 * _TCB), lambda i: (0, i))],
        out_specs=pl.BlockSpec((_TCB, 2 * EMBED_DIM), lambda i: (i, 0)),
        out_shape=jax.ShapeDtypeStruct((_RGRID * _TCB, 2 * EMBED_DIM), jnp.float32),
    )(tabt)


def _enc_body(out_ref):
    pos = lax.broadcasted_iota(jnp.int32, (SEQ_LEN, EMBED_DIM), 0).astype(jnp.float32)
    col = lax.broadcasted_iota(jnp.int32, (SEQ_LEN, EMBED_DIM), 1)
    expo = (2 * (col // 2)).astype(jnp.float32) / float(EMBED_DIM)
    ln_base = -math.log(MAX_WAVELENGTH)
    timescales = jnp.exp(expo * ln_base)
    angles = pos * timescales
    odd = (col % 2).astype(jnp.float32)
    out_ref[...] = jnp.sin(angles) * (1.0 - odd) + jnp.cos(angles) * odd


def _position_encoding_tc():
    return pl.pallas_call(
        _enc_body,
        out_shape=jax.ShapeDtypeStruct((SEQ_LEN, EMBED_DIM), jnp.float32),
    )()


_mesh = plsc.VectorSubcoreMesh(core_axis_name="c", subcore_axis_name="s")


@functools.partial(
    pl.kernel,
    out_type=jax.ShapeDtypeStruct((_ROWS, EMBED_DIM), jnp.float32),
    mesh=_mesh,
    scratch_types=[
        pltpu.VMEM((_BW,), jnp.int32),
        pltpu.VMEM((_BW,), jnp.int32),
        pltpu.VMEM((_CH, 2 * EMBED_DIM), jnp.float32),
        pltpu.VMEM((_CH, EMBED_DIM), jnp.float32),
        pltpu.VMEM((_CH, EMBED_DIM), jnp.float32),
        pltpu.SemaphoreType.DMA,
    ],
)
def _gather_add(tabr_hbm, idx_hbm, enc_hbm, out_hbm,
                idx_v, pidx_v, rows_v, enc_v, out_v, sem):
    wid = lax.axis_index("s") * _NC + lax.axis_index("c")
    base = wid * _BW
    enc_base = base % SEQ_LEN
    pltpu.sync_copy(idx_hbm.at[pl.ds(base, _BW)], idx_v)
    for j in range(_BW // 16):
        sl = pl.ds(j * 16, 16)
        v = idx_v[sl]
        pidx_v[sl] = lax.shift_left(lax.shift_right_logical(v, 11), 10) | (v & 1023)
    for ci in range(_NCHUNK):
        pltpu.async_copy(
            tabr_hbm.at[pidx_v.at[pl.ds(ci * _CH, _CH)]], rows_v, sem
        ).wait()
        pltpu.sync_copy(enc_hbm.at[pl.ds(enc_base + ci * _CH, _CH)], enc_v)

        def _add_block(b, carry):
            rbase = b * 16
            hi = (lax.shift_right_logical(
                idx_v[pl.ds(ci * _CH + rbase, 16)], 10) & 1) * EMBED_DIM
            for rr in range(16):
                off = hi[rr]
                r = rbase + rr
                for g in range(_LANES):
                    out_v[r, pl.ds(g * 16, 16)] = (
                        rows_v[r, pl.ds(off + g * 16, 16)]
                        + enc_v[r, pl.ds(g * 16, 16)]
                    )
            return carry

        lax.fori_loop(0, _CH // 16, _add_block, 0)
        pltpu.sync_copy(out_v, out_hbm.at[pl.ds(base + ci * _CH, _CH)])


def kernel(inputs, table):
    idx = inputs.reshape(-1).astype(jnp.int32)
    tabt = jnp.swapaxes(table, 0, 1)
    tabr = _relayout_tc(tabt)
    enc = _position_encoding_tc()
    out = _gather_add(tabr, idx, enc)
    return out.reshape(BATCH, SEQ_LEN, EMBED_DIM)


# speedup vs baseline: 1.5742x; 1.5742x over previous
import functools
import math

import jax
import jax.numpy as jnp
from jax import lax
from jax.experimental import pallas as pl
from jax.experimental.pallas import tpu as pltpu
from jax.experimental.pallas import tpu_sc as plsc

BATCH = 4
SEQ_LEN = 8192
EMBED_DIM = 64
VOCAB = 1000000
HALF_V = VOCAB // 2
MAX_WAVELENGTH = 10000.0

_NC = 2
_NS = 16
_NW = _NC * _NS
_ROWS = BATCH * SEQ_LEN
_BW = _ROWS // _NW
_CH = 256
_NCHUNK = _BW // _CH
_LANES = EMBED_DIM // 16

_TCB = 4096
_TCB_LOG2 = _TCB.bit_length() - 1


def _relayout_body(in_ref, out_ref):
    x = in_ref[...]
    r = lax.broadcasted_iota(jnp.int32, (EMBED_DIM, EMBED_DIM), 0)
    c = lax.broadcasted_iota(jnp.int32, (EMBED_DIM, EMBED_DIM), 1)
    ident = (r == c).astype(jnp.float32)
    xt = lax.dot_general(
        x, ident, (((0,), (0,)), ((), ())), preferred_element_type=jnp.float32
    )
    out_ref[...] = jnp.concatenate([xt[:_TCB], xt[_TCB:]], axis=1)


_RGRID = (VOCAB + 2 * _TCB - 1) // (2 * _TCB)


def _relayout_tc(tabt):
    return pl.pallas_call(
        _relayout_body,
        grid=(_RGRID,),
        in_specs=[pl.BlockSpec((EMBED_DIM, 2 * _TCB), lambda i: (0, i))],
        out_specs=pl.BlockSpec((_TCB, 2 * EMBED_DIM), lambda i: (i, 0)),
        out_shape=jax.ShapeDtypeStruct((_RGRID * _TCB, 2 * EMBED_DIM), jnp.float32),
    )(tabt)


def _enc_body(out_ref):
    pos = lax.broadcasted_iota(jnp.int32, (SEQ_LEN, EMBED_DIM), 0).astype(jnp.float32)
    col = lax.broadcasted_iota(jnp.int32, (SEQ_LEN, EMBED_DIM), 1)
    expo = (2 * (col // 2)).astype(jnp.float32) / float(EMBED_DIM)
    ln_base = -math.log(MAX_WAVELENGTH)
    timescales = jnp.exp(expo * ln_base)
    angles = pos * timescales
    odd = (col % 2).astype(jnp.float32)
    out_ref[...] = jnp.sin(angles) * (1.0 - odd) + jnp.cos(angles) * odd


def _position_encoding_tc():
    return pl.pallas_call(
        _enc_body,
        out_shape=jax.ShapeDtypeStruct((SEQ_LEN, EMBED_DIM), jnp.float32),
    )()


_mesh = plsc.VectorSubcoreMesh(core_axis_name="c", subcore_axis_name="s")


@functools.partial(
    pl.kernel,
    out_type=jax.ShapeDtypeStruct((_ROWS, EMBED_DIM), jnp.float32),
    mesh=_mesh,
    scratch_types=[
        pltpu.VMEM((_BW,), jnp.int32),
        pltpu.VMEM((_BW,), jnp.int32),
        pltpu.VMEM((_CH, 2 * EMBED_DIM), jnp.float32),
        pltpu.VMEM((_CH, EMBED_DIM), jnp.float32),
        pltpu.VMEM((_CH, EMBED_DIM), jnp.float32),
        pltpu.SemaphoreType.DMA,
    ],
)
def _gather_add(tabr_hbm, idx_hbm, enc_hbm, out_hbm,
                idx_v, pidx_v, rows_v, enc_v, out_v, sem):
    wid = lax.axis_index("s") * _NC + lax.axis_index("c")
    base = wid * _BW
    enc_base = base % SEQ_LEN
    pltpu.sync_copy(idx_hbm.at[pl.ds(base, _BW)], idx_v)
    for j in range(_BW // 16):
        sl = pl.ds(j * 16, 16)
        v = idx_v[sl]
        pidx_v[sl] = lax.shift_left(
            lax.shift_right_logical(v, _TCB_LOG2 + 1), _TCB_LOG2
        ) | (v & (_TCB - 1))
    for ci in range(_NCHUNK):
        pltpu.async_copy(
            tabr_hbm.at[pidx_v.at[pl.ds(ci * _CH, _CH)]], rows_v, sem
        ).wait()
        pltpu.sync_copy(enc_hbm.at[pl.ds(enc_base + ci * _CH, _CH)], enc_v)

        def _add_block(b, carry):
            rbase = b * 16
            hi = (lax.shift_right_logical(
                idx_v[pl.ds(ci * _CH + rbase, 16)], _TCB_LOG2) & 1) * EMBED_DIM
            for rr in range(16):
                off = hi[rr]
                r = rbase + rr
                for g in range(_LANES):
                    out_v[r, pl.ds(g * 16, 16)] = (
                        rows_v[r, pl.ds(off + g * 16, 16)]
                        + enc_v[r, pl.ds(g * 16, 16)]
                    )
            return carry

        lax.fori_loop(0, _CH // 16, _add_block, 0)
        pltpu.sync_copy(out_v, out_hbm.at[pl.ds(base + ci * _CH, _CH)])


def kernel(inputs, table):
    idx = inputs.reshape(-1).astype(jnp.int32)
    tabt = jnp.swapaxes(table, 0, 1)
    tabr = _relayout_tc(tabt)
    enc = _position_encoding_tc()
    out = _gather_add(tabr, idx, enc)
    return out.reshape(BATCH, SEQ_LEN, EMBED_DIM)


# speedup vs baseline: 1.7412x; 1.1061x over previous
import functools
import math

import jax
import jax.numpy as jnp
from jax import lax
from jax.experimental import pallas as pl
from jax.experimental.pallas import tpu as pltpu
from jax.experimental.pallas import tpu_sc as plsc

BATCH = 4
SEQ_LEN = 8192
EMBED_DIM = 64
VOCAB = 1000000
HALF_V = VOCAB // 2
MAX_WAVELENGTH = 10000.0

_NC = 2
_NS = 16
_NW = _NC * _NS
_ROWS = BATCH * SEQ_LEN
_BW = _ROWS // _NW
_CH = 256
_NCHUNK = _BW // _CH
_LANES = EMBED_DIM // 16

_TCB = 8192
_TCB_LOG2 = _TCB.bit_length() - 1


def _relayout_body(in_ref, out_ref):
    x = in_ref[...]
    r = lax.broadcasted_iota(jnp.int32, (EMBED_DIM, EMBED_DIM), 0)
    c = lax.broadcasted_iota(jnp.int32, (EMBED_DIM, EMBED_DIM), 1)
    ident = (r == c).astype(jnp.float32)
    xt = lax.dot_general(
        x, ident, (((0,), (0,)), ((), ())), preferred_element_type=jnp.float32
    )
    out_ref[...] = jnp.concatenate([xt[:_TCB], xt[_TCB:]], axis=1)


_RGRID = (VOCAB + 2 * _TCB - 1) // (2 * _TCB)


def _relayout_tc(tabt):
    return pl.pallas_call(
        _relayout_body,
        grid=(_RGRID,),
        in_specs=[pl.BlockSpec((EMBED_DIM, 2 * _TCB), lambda i: (0, i))],
        out_specs=pl.BlockSpec((_TCB, 2 * EMBED_DIM), lambda i: (i, 0)),
        out_shape=jax.ShapeDtypeStruct((_RGRID * _TCB, 2 * EMBED_DIM), jnp.float32),
    )(tabt)


def _enc_body(out_ref):
    pos = lax.broadcasted_iota(jnp.int32, (SEQ_LEN, EMBED_DIM), 0).astype(jnp.float32)
    col = lax.broadcasted_iota(jnp.int32, (SEQ_LEN, EMBED_DIM), 1)
    expo = (2 * (col // 2)).astype(jnp.float32) / float(EMBED_DIM)
    ln_base = -math.log(MAX_WAVELENGTH)
    timescales = jnp.exp(expo * ln_base)
    angles = pos * timescales
    odd = (col % 2).astype(jnp.float32)
    out_ref[...] = jnp.sin(angles) * (1.0 - odd) + jnp.cos(angles) * odd


def _position_encoding_tc():
    return pl.pallas_call(
        _enc_body,
        out_shape=jax.ShapeDtypeStruct((SEQ_LEN, EMBED_DIM), jnp.float32),
    )()


_mesh = plsc.VectorSubcoreMesh(core_axis_name="c", subcore_axis_name="s")


@functools.partial(
    pl.kernel,
    out_type=jax.ShapeDtypeStruct((_ROWS, EMBED_DIM), jnp.float32),
    mesh=_mesh,
    scratch_types=[
        pltpu.VMEM((_BW,), jnp.int32),
        pltpu.VMEM((_BW,), jnp.int32),
        pltpu.VMEM((_CH, 2 * EMBED_DIM), jnp.float32),
        pltpu.VMEM((_CH, EMBED_DIM), jnp.float32),
        pltpu.VMEM((_CH, EMBED_DIM), jnp.float32),
        pltpu.SemaphoreType.DMA,
    ],
)
def _gather_add(tabr_hbm, idx_hbm, enc_hbm, out_hbm,
                idx_v, pidx_v, rows_v, enc_v, out_v, sem):
    wid = lax.axis_index("s") * _NC + lax.axis_index("c")
    base = wid * _BW
    enc_base = base % SEQ_LEN
    pltpu.sync_copy(idx_hbm.at[pl.ds(base, _BW)], idx_v)
    for j in range(_BW // 16):
        sl = pl.ds(j * 16, 16)
        v = idx_v[sl]
        pidx_v[sl] = lax.shift_left(
            lax.shift_right_logical(v, _TCB_LOG2 + 1), _TCB_LOG2
        ) | (v & (_TCB - 1))
    for ci in range(_NCHUNK):
        pltpu.async_copy(
            tabr_hbm.at[pidx_v.at[pl.ds(ci * _CH, _CH)]], rows_v, sem
        ).wait()
        pltpu.sync_copy(enc_hbm.at[pl.ds(enc_base + ci * _CH, _CH)], enc_v)

        def _add_block(b, carry):
            rbase = b * 16
            hi = (lax.shift_right_logical(
                idx_v[pl.ds(ci * _CH + rbase, 16)], _TCB_LOG2) & 1) * EMBED_DIM
            for rr in range(16):
                off = hi[rr]
                r = rbase + rr
                for g in range(_LANES):
                    out_v[r, pl.ds(g * 16, 16)] = (
                        rows_v[r, pl.ds(off + g * 16, 16)]
                        + enc_v[r, pl.ds(g * 16, 16)]
                    )
            return carry

        lax.fori_loop(0, _CH // 16, _add_block, 0)
        pltpu.sync_copy(out_v, out_hbm.at[pl.ds(base + ci * _CH, _CH)])


def kernel(inputs, table):
    idx = inputs.reshape(-1).astype(jnp.int32)
    tabt = jnp.swapaxes(table, 0, 1)
    tabr = _relayout_tc(tabt)
    enc = _position_encoding_tc()
    out = _gather_add(tabr, idx, enc)
    return out.reshape(BATCH, SEQ_LEN, EMBED_DIM)


# speedup vs baseline: 1.8285x; 1.0501x over previous
import functools
import math

import jax
import jax.numpy as jnp
from jax import lax
from jax.experimental import pallas as pl
from jax.experimental.pallas import tpu as pltpu
from jax.experimental.pallas import tpu_sc as plsc

BATCH = 4
SEQ_LEN = 8192
EMBED_DIM = 64
VOCAB = 1000000
HALF_V = VOCAB // 2
MAX_WAVELENGTH = 10000.0

_NC = 2
_NS = 16
_NW = _NC * _NS
_ROWS = BATCH * SEQ_LEN
_BW = _ROWS // _NW
_CH = 128
_NCHUNK = _BW // _CH
_LANES = EMBED_DIM // 16

_TCB = 8192
_TCB_LOG2 = _TCB.bit_length() - 1


def _relayout_body(in_ref, out_ref):
    x = in_ref[...]
    r = lax.broadcasted_iota(jnp.int32, (EMBED_DIM, EMBED_DIM), 0)
    c = lax.broadcasted_iota(jnp.int32, (EMBED_DIM, EMBED_DIM), 1)
    ident = (r == c).astype(jnp.float32)
    dn = (((0,), (0,)), ((), ()))
    out_ref[:, :EMBED_DIM] = lax.dot_general(
        x[:, :_TCB], ident, dn, preferred_element_type=jnp.float32
    )
    out_ref[:, EMBED_DIM:] = lax.dot_general(
        x[:, _TCB:], ident, dn, preferred_element_type=jnp.float32
    )


_RGRID = (VOCAB + 2 * _TCB - 1) // (2 * _TCB)


def _relayout_tc(tabt):
    return pl.pallas_call(
        _relayout_body,
        grid=(_RGRID,),
        in_specs=[pl.BlockSpec((EMBED_DIM, 2 * _TCB), lambda i: (0, i))],
        out_specs=pl.BlockSpec((_TCB, 2 * EMBED_DIM), lambda i: (i, 0)),
        out_shape=jax.ShapeDtypeStruct((_RGRID * _TCB, 2 * EMBED_DIM), jnp.float32),
    )(tabt)


def _enc_body(out_ref):
    pos = lax.broadcasted_iota(jnp.int32, (SEQ_LEN, EMBED_DIM), 0).astype(jnp.float32)
    col = lax.broadcasted_iota(jnp.int32, (SEQ_LEN, EMBED_DIM), 1)
    expo = (2 * (col // 2)).astype(jnp.float32) / float(EMBED_DIM)
    ln_base = -math.log(MAX_WAVELENGTH)
    timescales = jnp.exp(expo * ln_base)
    angles = pos * timescales
    odd = (col % 2).astype(jnp.float32)
    out_ref[...] = jnp.sin(angles) * (1.0 - odd) + jnp.cos(angles) * odd


def _position_encoding_tc():
    return pl.pallas_call(
        _enc_body,
        out_shape=jax.ShapeDtypeStruct((SEQ_LEN, EMBED_DIM), jnp.float32),
    )()


_mesh = plsc.VectorSubcoreMesh(core_axis_name="c", subcore_axis_name="s")


@functools.partial(
    pl.kernel,
    out_type=jax.ShapeDtypeStruct((_ROWS, EMBED_DIM), jnp.float32),
    mesh=_mesh,
    scratch_types=[
        pltpu.VMEM((_BW,), jnp.int32),
        pltpu.VMEM((_BW,), jnp.int32),
        pltpu.VMEM((2, _CH, 2 * EMBED_DIM), jnp.float32),
        pltpu.VMEM((2, _CH, EMBED_DIM), jnp.float32),
        pltpu.VMEM((2, _CH, EMBED_DIM), jnp.float32),
        pltpu.SemaphoreType.DMA,
        pltpu.SemaphoreType.DMA,
        pltpu.SemaphoreType.DMA,
        pltpu.SemaphoreType.DMA,
    ],
)
def _gather_add(tabr_hbm, idx_hbm, enc_hbm, out_hbm,
                idx_v, pidx_v, rows_v, enc_v, out_v,
                isem0, isem1, osem0, osem1):
    wid = lax.axis_index("s") * _NC + lax.axis_index("c")
    base = wid * _BW
    enc_base = base % SEQ_LEN
    isems = (isem0, isem1)
    osems = (osem0, osem1)
    pltpu.sync_copy(idx_hbm.at[pl.ds(base, _BW)], idx_v)
    for j in range(_BW // 16):
        sl = pl.ds(j * 16, 16)
        v = idx_v[sl]
        pidx_v[sl] = lax.shift_left(
            lax.shift_right_logical(v, _TCB_LOG2 + 1), _TCB_LOG2
        ) | (v & (_TCB - 1))

    def _fire(ci):
        b = ci % 2
        gd = pltpu.async_copy(
            tabr_hbm.at[pidx_v.at[pl.ds(ci * _CH, _CH)]], rows_v.at[b], isems[b]
        )
        ed = pltpu.async_copy(
            enc_hbm.at[pl.ds(enc_base + ci * _CH, _CH)], enc_v.at[b], isems[b]
        )
        return gd, ed

    pending = {0: _fire(0)}
    out_pending = {}
    for ci in range(_NCHUNK):
        b = ci % 2
        if ci + 1 < _NCHUNK:
            pending[ci + 1] = _fire(ci + 1)
        if ci - 2 in out_pending:
            out_pending.pop(ci - 2).wait()
        gd, ed = pending.pop(ci)
        gd.wait()
        ed.wait()

        def _add_block(blk, carry, ci=ci, b=b):
            rbase = blk * 16
            hi = (lax.shift_right_logical(
                idx_v[pl.ds(ci * _CH + rbase, 16)], _TCB_LOG2) & 1) * EMBED_DIM
            for rr in range(16):
                off = hi[rr]
                r = rbase + rr
                for g in range(_LANES):
                    out_v[b, r, pl.ds(g * 16, 16)] = (
                        rows_v[b, r, pl.ds(off + g * 16, 16)]
                        + enc_v[b, r, pl.ds(g * 16, 16)]
                    )
            return carry

        lax.fori_loop(0, _CH // 16, _add_block, 0)
        out_pending[ci] = pltpu.async_copy(
            out_v.at[b], out_hbm.at[pl.ds(base + ci * _CH, _CH)], osems[b]
        )
    for d in out_pending.values():
        d.wait()


def kernel(inputs, table):
    idx = inputs.reshape(-1).astype(jnp.int32)
    tabt = jnp.swapaxes(table, 0, 1)
    tabr = _relayout_tc(tabt)
    enc = _position_encoding_tc()
    out = _gather_add(tabr, idx, enc)
    return out.reshape(BATCH, SEQ_LEN, EMBED_DIM)


# speedup vs baseline: 1.8303x; 1.0010x over previous
import functools
import math

import jax
import jax.numpy as jnp
from jax import lax
from jax.experimental import pallas as pl
from jax.experimental.pallas import tpu as pltpu
from jax.experimental.pallas import tpu_sc as plsc

BATCH = 4
SEQ_LEN = 8192
EMBED_DIM = 64
VOCAB = 1000000
HALF_V = VOCAB // 2
MAX_WAVELENGTH = 10000.0

_NC = 2
_NS = 16
_NW = _NC * _NS
_ROWS = BATCH * SEQ_LEN
_BW = _ROWS // _NW
_CH = 128
_NCHUNK = _BW // _CH
_LANES = EMBED_DIM // 16

_TCB = 8192
_TCB_LOG2 = _TCB.bit_length() - 1


def _relayout_body(in_ref, out_ref):
    x = in_ref[...]
    r = lax.broadcasted_iota(jnp.int32, (EMBED_DIM, EMBED_DIM), 0)
    c = lax.broadcasted_iota(jnp.int32, (EMBED_DIM, EMBED_DIM), 1)
    ident = (r == c).astype(jnp.float32)
    dn = (((0,), (0,)), ((), ()))
    out_ref[:, :EMBED_DIM] = lax.dot_general(
        x[:, :_TCB], ident, dn, preferred_element_type=jnp.float32
    )
    out_ref[:, EMBED_DIM:] = lax.dot_general(
        x[:, _TCB:], ident, dn, preferred_element_type=jnp.float32
    )


_NBLK_ALL = (VOCAB + 2 * _TCB - 1) // (2 * _TCB)


def _relayout_tc(tabt):
    return pl.pallas_call(
        _relayout_body,
        grid=(_NBLK_ALL,),
        in_specs=[pl.BlockSpec((EMBED_DIM, 2 * _TCB), lambda i: (0, i))],
        out_specs=pl.BlockSpec((_TCB, 2 * EMBED_DIM), lambda i: (i, 0)),
        out_shape=jax.ShapeDtypeStruct((_NBLK_ALL * _TCB, 2 * EMBED_DIM), jnp.float32),
    )(tabt)


def _enc_body(out_ref):
    pos = lax.broadcasted_iota(jnp.int32, (SEQ_LEN, EMBED_DIM), 0).astype(jnp.float32)
    col = lax.broadcasted_iota(jnp.int32, (SEQ_LEN, EMBED_DIM), 1)
    expo = (2 * (col // 2)).astype(jnp.float32) / float(EMBED_DIM)
    ln_base = -math.log(MAX_WAVELENGTH)
    timescales = jnp.exp(expo * ln_base)
    angles = pos * timescales
    odd = (col % 2).astype(jnp.float32)
    out_ref[...] = jnp.sin(angles) * (1.0 - odd) + jnp.cos(angles) * odd


def _position_encoding_tc():
    return pl.pallas_call(
        _enc_body,
        out_shape=jax.ShapeDtypeStruct((SEQ_LEN, EMBED_DIM), jnp.float32),
    )()


_mesh = plsc.VectorSubcoreMesh(core_axis_name="c", subcore_axis_name="s")


@functools.partial(
    pl.kernel,
    out_type=jax.ShapeDtypeStruct((_ROWS, EMBED_DIM), jnp.float32),
    mesh=_mesh,
    scratch_types=[
        pltpu.VMEM((_BW,), jnp.int32),
        pltpu.VMEM((_BW,), jnp.int32),
        pltpu.VMEM((2, _CH, 2 * EMBED_DIM), jnp.float32),
        pltpu.VMEM((2, _CH, EMBED_DIM), jnp.float32),
        pltpu.VMEM((2, _CH, EMBED_DIM), jnp.float32),
        pltpu.SemaphoreType.DMA,
        pltpu.SemaphoreType.DMA,
        pltpu.SemaphoreType.DMA,
        pltpu.SemaphoreType.DMA,
    ],
)
def _gather_add(tabr_hbm, idx_hbm, enc_hbm, out_hbm,
                idx_v, pidx_v, rows_v, enc_v, out_v,
                isem0, isem1, osem0, osem1):
    wid = lax.axis_index("s") * _NC + lax.axis_index("c")
    base = wid * _BW
    enc_base = base % SEQ_LEN
    isems = (isem0, isem1)
    osems = (osem0, osem1)
    pltpu.sync_copy(idx_hbm.at[pl.ds(base, _BW)], idx_v)
    for j in range(_BW // 16):
        sl = pl.ds(j * 16, 16)
        v = idx_v[sl]
        pidx_v[sl] = lax.shift_left(
            lax.shift_right_logical(v, _TCB_LOG2 + 1), _TCB_LOG2
        ) | (v & (_TCB - 1))

    def _fire(ci):
        b = ci % 2
        sl = pl.ds(ci * _CH, _CH)
        gd = pltpu.async_copy(
            tabr_hbm.at[pidx_v.at[sl]], rows_v.at[b], isems[b]
        )
        ed = pltpu.async_copy(
            enc_hbm.at[pl.ds(enc_base + ci * _CH, _CH)], enc_v.at[b], isems[b]
        )
        return gd, ed

    pending = {0: _fire(0)}
    out_pending = {}
    for ci in range(_NCHUNK):
        b = ci % 2
        if ci + 1 < _NCHUNK:
            pending[ci + 1] = _fire(ci + 1)
        if ci - 2 in out_pending:
            out_pending.pop(ci - 2).wait()
        for d in pending.pop(ci):
            d.wait()

        def _add_block(blk, carry, ci=ci, b=b):
            rbase = blk * 16
            v16 = idx_v[pl.ds(ci * _CH + rbase, 16)]
            hi = (lax.shift_right_logical(v16, _TCB_LOG2) & 1) * EMBED_DIM
            for rr in range(16):
                off = hi[rr]
                r = rbase + rr
                for g in range(_LANES):
                    out_v[b, r, pl.ds(g * 16, 16)] = (
                        rows_v[b, r, pl.ds(off + g * 16, 16)]
                        + enc_v[b, r, pl.ds(g * 16, 16)]
                    )
            return carry

        lax.fori_loop(0, _CH // 16, _add_block, 0)
        out_pending[ci] = pltpu.async_copy(
            out_v.at[b], out_hbm.at[pl.ds(base + ci * _CH, _CH)], osems[b]
        )
    for d in out_pending.values():
        d.wait()


def kernel(inputs, table):
    idx = inputs.reshape(-1).astype(jnp.int32)
    tabt = jnp.swapaxes(table, 0, 1)
    tabr = _relayout_tc(tabt)
    enc = _position_encoding_tc()
    out = _gather_add(tabr, idx, enc)
    return out.reshape(BATCH, SEQ_LEN, EMBED_DIM)


# speedup vs baseline: 1.8319x; 1.0009x over previous
import functools
import math

import jax
import jax.numpy as jnp
from jax import lax
from jax.experimental import pallas as pl
from jax.experimental.pallas import tpu as pltpu
from jax.experimental.pallas import tpu_sc as plsc

BATCH = 4
SEQ_LEN = 8192
EMBED_DIM = 64
VOCAB = 1000000
HALF_V = VOCAB // 2
MAX_WAVELENGTH = 10000.0

_NC = 2
_NS = 16
_NW = _NC * _NS
_ROWS = BATCH * SEQ_LEN
_BW = _ROWS // _NW
_CH = 128
_NCHUNK = _BW // _CH
_LANES = EMBED_DIM // 16

_TCB = 8192
_TCB_LOG2 = _TCB.bit_length() - 1


def _relayout_body(in_ref, out_ref):
    x = in_ref[...]
    r = lax.broadcasted_iota(jnp.int32, (EMBED_DIM, EMBED_DIM), 0)
    c = lax.broadcasted_iota(jnp.int32, (EMBED_DIM, EMBED_DIM), 1)
    ident = (r == c).astype(jnp.float32)
    dn = (((0,), (0,)), ((), ()))
    out_ref[:, :EMBED_DIM] = lax.dot_general(
        x[:, :_TCB], ident, dn, preferred_element_type=jnp.float32
    )
    out_ref[:, EMBED_DIM:] = lax.dot_general(
        x[:, _TCB:], ident, dn, preferred_element_type=jnp.float32
    )


_NBLK_ALL = (VOCAB + 2 * _TCB - 1) // (2 * _TCB)


def _relayout_tc(tabt):
    return pl.pallas_call(
        _relayout_body,
        grid=(_NBLK_ALL,),
        in_specs=[pl.BlockSpec((EMBED_DIM, 2 * _TCB), lambda i: (0, i))],
        out_specs=pl.BlockSpec((_TCB, 2 * EMBED_DIM), lambda i: (i, 0)),
        out_shape=jax.ShapeDtypeStruct((_NBLK_ALL * _TCB, 2 * EMBED_DIM), jnp.float32),
    )(tabt)


def _enc_body(out_ref):
    pos = lax.broadcasted_iota(jnp.int32, (SEQ_LEN, EMBED_DIM), 0).astype(jnp.float32)
    col = lax.broadcasted_iota(jnp.int32, (SEQ_LEN, EMBED_DIM), 1)
    expo = (2 * (col // 2)).astype(jnp.float32) / float(EMBED_DIM)
    ln_base = -math.log(MAX_WAVELENGTH)
    timescales = jnp.exp(expo * ln_base)
    angles = pos * timescales
    odd = (col % 2).astype(jnp.float32)
    out_ref[...] = jnp.sin(angles + odd * (math.pi / 2.0))


def _position_encoding_tc():
    return pl.pallas_call(
        _enc_body,
        out_shape=jax.ShapeDtypeStruct((SEQ_LEN, EMBED_DIM), jnp.float32),
    )()


_mesh = plsc.VectorSubcoreMesh(core_axis_name="c", subcore_axis_name="s")


@functools.partial(
    pl.kernel,
    out_type=jax.ShapeDtypeStruct((_ROWS, EMBED_DIM), jnp.float32),
    mesh=_mesh,
    scratch_types=[
        pltpu.VMEM((_BW,), jnp.int32),
        pltpu.VMEM((_BW,), jnp.int32),
        pltpu.VMEM((2, _CH, 2 * EMBED_DIM), jnp.float32),
        pltpu.VMEM((2, _CH, EMBED_DIM), jnp.float32),
        pltpu.VMEM((2, _CH, EMBED_DIM), jnp.float32),
        pltpu.SemaphoreType.DMA,
        pltpu.SemaphoreType.DMA,
        pltpu.SemaphoreType.DMA,
        pltpu.SemaphoreType.DMA,
    ],
)
def _gather_add(tabr_hbm, idx_hbm, enc_hbm, out_hbm,
                idx_v, pidx_v, rows_v, enc_v, out_v,
                isem0, isem1, osem0, osem1):
    wid = lax.axis_index("s") * _NC + lax.axis_index("c")
    base = wid * _BW
    enc_base = base % SEQ_LEN
    isems = (isem0, isem1)
    osems = (osem0, osem1)
    pltpu.sync_copy(idx_hbm.at[pl.ds(base, _BW)], idx_v)
    for j in range(_BW // 16):
        sl = pl.ds(j * 16, 16)
        v = idx_v[sl]
        pidx_v[sl] = lax.shift_left(
            lax.shift_right_logical(v, _TCB_LOG2 + 1), _TCB_LOG2
        ) | (v & (_TCB - 1))

    def _fire(ci):
        b = ci % 2
        sl = pl.ds(ci * _CH, _CH)
        gd = pltpu.async_copy(
            tabr_hbm.at[pidx_v.at[sl]], rows_v.at[b], isems[b]
        )
        ed = pltpu.async_copy(
            enc_hbm.at[pl.ds(enc_base + ci * _CH, _CH)], enc_v.at[b], isems[b]
        )
        return gd, ed

    pending = {0: _fire(0)}
    out_pending = {}
    for ci in range(_NCHUNK):
        b = ci % 2
        if ci + 1 < _NCHUNK:
            pending[ci + 1] = _fire(ci + 1)
        if ci - 2 in out_pending:
            out_pending.pop(ci - 2).wait()
        for d in pending.pop(ci):
            d.wait()

        def _add_block(blk, carry, ci=ci, b=b):
            rbase = blk * 16
            v16 = idx_v[pl.ds(ci * _CH + rbase, 16)]
            hi = (lax.shift_right_logical(v16, _TCB_LOG2) & 1) * EMBED_DIM
            for rr in range(16):
                off = hi[rr]
                r = rbase + rr
                for g in range(_LANES):
                    out_v[b, r, pl.ds(g * 16, 16)] = (
                        rows_v[b, r, pl.ds(off + g * 16, 16)]
                        + enc_v[b, r, pl.ds(g * 16, 16)]
                    )
            return carry

        lax.fori_loop(0, _CH // 16, _add_block, 0)
        out_pending[ci] = pltpu.async_copy(
            out_v.at[b], out_hbm.at[pl.ds(base + ci * _CH, _CH)], osems[b]
        )
    for d in out_pending.values():
        d.wait()


def kernel(inputs, table):
    idx = inputs.reshape(-1).astype(jnp.int32)
    tabt = jnp.swapaxes(table, 0, 1)
    tabr = _relayout_tc(tabt)
    enc = _position_encoding_tc()
    out = _gather_add(tabr, idx, enc)
    return out.reshape(BATCH, SEQ_LEN, EMBED_DIM)


# speedup vs baseline: 2.0596x; 1.1243x over previous
import functools
import math

import jax
import jax.numpy as jnp
from jax import lax
from jax.experimental import pallas as pl
from jax.experimental.pallas import tpu as pltpu
from jax.experimental.pallas import tpu_sc as plsc

BATCH = 4
SEQ_LEN = 8192
EMBED_DIM = 64
VOCAB = 1000000
HALF_V = VOCAB // 2
MAX_WAVELENGTH = 10000.0

_NC = 2
_NS = 16
_NW = _NC * _NS
_ROWS = BATCH * SEQ_LEN
_BW = _ROWS // _NW
_CH = 128
_NCHUNK = _BW // _CH
_LANES = EMBED_DIM // 16

_TCB = 8192
_TCB_LOG2 = _TCB.bit_length() - 1


def _relayout_body(in_ref, out_ref):
    x = in_ref[...]
    r = lax.broadcasted_iota(jnp.int32, (EMBED_DIM, EMBED_DIM), 0)
    c = lax.broadcasted_iota(jnp.int32, (EMBED_DIM, EMBED_DIM), 1)
    ident = (r == c).astype(jnp.bfloat16)
    xb = x.astype(jnp.bfloat16)
    dn = (((0,), (0,)), ((), ()))
    out_ref[:, :EMBED_DIM] = lax.dot_general(
        xb[:, :_TCB], ident, dn, preferred_element_type=jnp.float32
    )
    out_ref[:, EMBED_DIM:] = lax.dot_general(
        xb[:, _TCB:], ident, dn, preferred_element_type=jnp.float32
    )


_NBLK_ALL = (VOCAB + 2 * _TCB - 1) // (2 * _TCB)


def _relayout_tc(tabt):
    return pl.pallas_call(
        _relayout_body,
        grid=(_NBLK_ALL,),
        in_specs=[pl.BlockSpec((EMBED_DIM, 2 * _TCB), lambda i: (0, i))],
        out_specs=pl.BlockSpec((_TCB, 2 * EMBED_DIM), lambda i: (i, 0)),
        out_shape=jax.ShapeDtypeStruct((_NBLK_ALL * _TCB, 2 * EMBED_DIM), jnp.float32),
    )(tabt)


def _enc_body(out_ref):
    pos = lax.broadcasted_iota(jnp.int32, (SEQ_LEN, EMBED_DIM), 0).astype(jnp.float32)
    col = lax.broadcasted_iota(jnp.int32, (SEQ_LEN, EMBED_DIM), 1)
    expo = (2 * (col // 2)).astype(jnp.float32) / float(EMBED_DIM)
    ln_base = -math.log(MAX_WAVELENGTH)
    timescales = jnp.exp(expo * ln_base)
    angles = pos * timescales
    odd = (col % 2).astype(jnp.float32)
    out_ref[...] = jnp.sin(angles + odd * (math.pi / 2.0))


def _position_encoding_tc():
    return pl.pallas_call(
        _enc_body,
        out_shape=jax.ShapeDtypeStruct((SEQ_LEN, EMBED_DIM), jnp.float32),
    )()


_mesh = plsc.VectorSubcoreMesh(core_axis_name="c", subcore_axis_name="s")


@functools.partial(
    pl.kernel,
    out_type=jax.ShapeDtypeStruct((_ROWS, EMBED_DIM), jnp.float32),
    mesh=_mesh,
    scratch_types=[
        pltpu.VMEM((_BW,), jnp.int32),
        pltpu.VMEM((_BW,), jnp.int32),
        pltpu.VMEM((2, _CH, 2 * EMBED_DIM), jnp.float32),
        pltpu.VMEM((2, _CH, EMBED_DIM), jnp.float32),
        pltpu.VMEM((2, _CH, EMBED_DIM), jnp.float32),
        pltpu.SemaphoreType.DMA,
        pltpu.SemaphoreType.DMA,
        pltpu.SemaphoreType.DMA,
        pltpu.SemaphoreType.DMA,
    ],
)
def _gather_add(tabr_hbm, idx_hbm, enc_hbm, out_hbm,
                idx_v, pidx_v, rows_v, enc_v, out_v,
                isem0, isem1, osem0, osem1):
    wid = lax.axis_index("s") * _NC + lax.axis_index("c")
    base = wid * _BW
    enc_base = base % SEQ_LEN
    isems = (isem0, isem1)
    osems = (osem0, osem1)
    pltpu.sync_copy(idx_hbm.at[pl.ds(base, _BW)], idx_v)
    for j in range(_BW // 16):
        sl = pl.ds(j * 16, 16)
        v = idx_v[sl]
        pidx_v[sl] = lax.shift_left(
            lax.shift_right_logical(v, _TCB_LOG2 + 1), _TCB_LOG2
        ) | (v & (_TCB - 1))

    def _fire(ci):
        b = ci % 2
        sl = pl.ds(ci * _CH, _CH)
        gd = pltpu.async_copy(
            tabr_hbm.at[pidx_v.at[sl]], rows_v.at[b], isems[b]
        )
        ed = pltpu.async_copy(
            enc_hbm.at[pl.ds(enc_base + ci * _CH, _CH)], enc_v.at[b], isems[b]
        )
        return gd, ed

    pending = {0: _fire(0)}
    out_pending = {}
    for ci in range(_NCHUNK):
        b = ci % 2
        if ci + 1 < _NCHUNK:
            pending[ci + 1] = _fire(ci + 1)
        if ci - 2 in out_pending:
            out_pending.pop(ci - 2).wait()
        for d in pending.pop(ci):
            d.wait()

        def _add_block(blk, carry, ci=ci, b=b):
            rbase = blk * 16
            v16 = idx_v[pl.ds(ci * _CH + rbase, 16)]
            hi = (lax.shift_right_logical(v16, _TCB_LOG2) & 1) * EMBED_DIM
            for rr in range(16):
                off = hi[rr]
                r = rbase + rr
                for g in range(_LANES):
                    out_v[b, r, pl.ds(g * 16, 16)] = (
                        rows_v[b, r, pl.ds(off + g * 16, 16)]
                        + enc_v[b, r, pl.ds(g * 16, 16)]
                    )
            return carry

        lax.fori_loop(0, _CH // 16, _add_block, 0)
        out_pending[ci] = pltpu.async_copy(
            out_v.at[b], out_hbm.at[pl.ds(base + ci * _CH, _CH)], osems[b]
        )
    for d in out_pending.values():
        d.wait()


def kernel(inputs, table):
    idx = inputs.reshape(-1).astype(jnp.int32)
    tabt = jnp.swapaxes(table, 0, 1)
    tabr = _relayout_tc(tabt)
    enc = _position_encoding_tc()
    out = _gather_add(tabr, idx, enc)
    return out.reshape(BATCH, SEQ_LEN, EMBED_DIM)


# speedup vs baseline: 2.2106x; 1.0733x over previous
import functools
import math

import jax
import jax.numpy as jnp
from jax import lax
from jax.experimental import pallas as pl
from jax.experimental.pallas import tpu as pltpu
from jax.experimental.pallas import tpu_sc as plsc

BATCH = 4
SEQ_LEN = 8192
EMBED_DIM = 64
VOCAB = 1000000
HALF_V = VOCAB // 2
MAX_WAVELENGTH = 10000.0

_NC = 2
_NS = 16
_NW = _NC * _NS
_ROWS = BATCH * SEQ_LEN
_BW = _ROWS // _NW
_CH = 128
_NCHUNK = _BW // _CH
_LANES = EMBED_DIM // 16

_TCB = 16384
_TCB_LOG2 = _TCB.bit_length() - 1


def _relayout_body(in_ref, out_ref):
    x = in_ref[...]
    r = lax.broadcasted_iota(jnp.int32, (EMBED_DIM, EMBED_DIM), 0)
    c = lax.broadcasted_iota(jnp.int32, (EMBED_DIM, EMBED_DIM), 1)
    ident = (r == c).astype(jnp.bfloat16)
    xb = x.astype(jnp.bfloat16)
    dn = (((0,), (0,)), ((), ()))
    out_ref[:, :EMBED_DIM] = lax.dot_general(
        xb[:, :_TCB], ident, dn, preferred_element_type=jnp.float32
    )
    out_ref[:, EMBED_DIM:] = lax.dot_general(
        xb[:, _TCB:], ident, dn, preferred_element_type=jnp.float32
    )


_NBLK_ALL = (VOCAB + 2 * _TCB - 1) // (2 * _TCB)


def _relayout_tc(tabt):
    return pl.pallas_call(
        _relayout_body,
        grid=(_NBLK_ALL,),
        in_specs=[pl.BlockSpec((EMBED_DIM, 2 * _TCB), lambda i: (0, i))],
        out_specs=pl.BlockSpec((_TCB, 2 * EMBED_DIM), lambda i: (i, 0)),
        out_shape=jax.ShapeDtypeStruct((_NBLK_ALL * _TCB, 2 * EMBED_DIM), jnp.float32),
    )(tabt)


def _enc_body(out_ref):
    pos = lax.broadcasted_iota(jnp.int32, (SEQ_LEN, EMBED_DIM), 0).astype(jnp.float32)
    col = lax.broadcasted_iota(jnp.int32, (SEQ_LEN, EMBED_DIM), 1)
    expo = (2 * (col // 2)).astype(jnp.float32) / float(EMBED_DIM)
    ln_base = -math.log(MAX_WAVELENGTH)
    timescales = jnp.exp(expo * ln_base)
    angles = pos * timescales
    odd = (col % 2).astype(jnp.float32)
    out_ref[...] = jnp.sin(angles + odd * (math.pi / 2.0))


def _position_encoding_tc():
    return pl.pallas_call(
        _enc_body,
        out_shape=jax.ShapeDtypeStruct((SEQ_LEN, EMBED_DIM), jnp.float32),
    )()


_mesh = plsc.VectorSubcoreMesh(core_axis_name="c", subcore_axis_name="s")


@functools.partial(
    pl.kernel,
    out_type=jax.ShapeDtypeStruct((_ROWS, EMBED_DIM), jnp.float32),
    mesh=_mesh,
    scratch_types=[
        pltpu.VMEM((_BW,), jnp.int32),
        pltpu.VMEM((_BW,), jnp.int32),
        pltpu.VMEM((2, _CH, 2 * EMBED_DIM), jnp.float32),
        pltpu.VMEM((2, _CH, EMBED_DIM), jnp.float32),
        pltpu.VMEM((2, _CH, EMBED_DIM), jnp.float32),
        pltpu.SemaphoreType.DMA,
        pltpu.SemaphoreType.DMA,
        pltpu.SemaphoreType.DMA,
        pltpu.SemaphoreType.DMA,
    ],
)
def _gather_add(tabr_hbm, idx_hbm, enc_hbm, out_hbm,
                idx_v, pidx_v, rows_v, enc_v, out_v,
                isem0, isem1, osem0, osem1):
    wid = lax.axis_index("s") * _NC + lax.axis_index("c")
    base = wid * _BW
    enc_base = base % SEQ_LEN
    isems = (isem0, isem1)
    osems = (osem0, osem1)
    pltpu.sync_copy(idx_hbm.at[pl.ds(base, _BW)], idx_v)
    for j in range(_BW // 16):
        sl = pl.ds(j * 16, 16)
        v = idx_v[sl]
        pidx_v[sl] = lax.shift_left(
            lax.shift_right_logical(v, _TCB_LOG2 + 1), _TCB_LOG2
        ) | (v & (_TCB - 1))

    def _fire(ci):
        b = ci % 2
        sl = pl.ds(ci * _CH, _CH)
        gd = pltpu.async_copy(
            tabr_hbm.at[pidx_v.at[sl]], rows_v.at[b], isems[b]
        )
        ed = pltpu.async_copy(
            enc_hbm.at[pl.ds(enc_base + ci * _CH, _CH)], enc_v.at[b], isems[b]
        )
        return gd, ed

    pending = {0: _fire(0)}
    out_pending = {}
    for ci in range(_NCHUNK):
        b = ci % 2
        if ci + 1 < _NCHUNK:
            pending[ci + 1] = _fire(ci + 1)
        if ci - 2 in out_pending:
            out_pending.pop(ci - 2).wait()
        for d in pending.pop(ci):
            d.wait()

        def _add_block(blk, carry, ci=ci, b=b):
            rbase = blk * 16
            v16 = idx_v[pl.ds(ci * _CH + rbase, 16)]
            hi = (lax.shift_right_logical(v16, _TCB_LOG2) & 1) * EMBED_DIM
            for rr in range(16):
                off = hi[rr]
                r = rbase + rr
                for g in range(_LANES):
                    out_v[b, r, pl.ds(g * 16, 16)] = (
                        rows_v[b, r, pl.ds(off + g * 16, 16)]
                        + enc_v[b, r, pl.ds(g * 16, 16)]
                    )
            return carry

        lax.fori_loop(0, _CH // 16, _add_block, 0)
        out_pending[ci] = pltpu.async_copy(
            out_v.at[b], out_hbm.at[pl.ds(base + ci * _CH, _CH)], osems[b]
        )
    for d in out_pending.values():
        d.wait()


def kernel(inputs, table):
    idx = inputs.reshape(-1).astype(jnp.int32)
    tabt = jnp.swapaxes(table, 0, 1)
    tabr = _relayout_tc(tabt)
    enc = _position_encoding_tc()
    out = _gather_add(tabr, idx, enc)
    return out.reshape(BATCH, SEQ_LEN, EMBED_DIM)
